# baseline (speedup 1.0000x reference)
import functools

import jax
import jax.numpy as jnp
from jax import lax
from jax.experimental import pallas as pl
from jax.experimental.pallas import tpu as pltpu
from jax.experimental.pallas import tpu_sc as plsc

_LANES = 16
_CHUNK = 128


@functools.lru_cache(maxsize=None)
def _make_kernel(B, D):
    info = plsc.get_sparse_core_info()
    NC, NS = info.num_cores, info.num_subcores
    NW = NC * NS
    bpw = B // NW
    nchunk = bpw // _CHUNK

    mesh = plsc.VectorSubcoreMesh(core_axis_name="c", subcore_axis_name="s")

    @functools.partial(
        pl.kernel,
        mesh=mesh,
        out_type=jax.ShapeDtypeStruct((B,), jnp.float32),
        compiler_params=pltpu.CompilerParams(
            needs_layout_passes=False, use_tc_tiling_on_sc=False
        ),
        scratch_types=[
            pltpu.VMEM((nchunk, _CHUNK), jnp.int32),
            pltpu.VMEM((nchunk, _CHUNK), jnp.int32),
            pltpu.VMEM((bpw, D), jnp.float32),
            pltpu.VMEM((bpw, D), jnp.float32),
            pltpu.VMEM((bpw,), jnp.float32),
            pltpu.SemaphoreType.DMA,
        ],
    )
    def kern(uidx_hbm, iidx_hbm, ut_hbm, it_hbm, out_hbm,
             uix, iix, urows, irows, outv, sem):
        wid = lax.axis_index("s") * NC + lax.axis_index("c")
        base = wid * bpw
        crow = wid * nchunk
        pltpu.sync_copy(uidx_hbm.at[pl.ds(crow, nchunk)], uix)
        pltpu.sync_copy(iidx_hbm.at[pl.ds(crow, nchunk)], iix)
        copies = []
        for j in range(nchunk):
            dst = pl.ds(j * _CHUNK, _CHUNK)
            copies.append(pltpu.async_copy(ut_hbm.at[uix.at[j]], urows.at[dst], sem))
            copies.append(pltpu.async_copy(it_hbm.at[iix.at[j]], irows.at[dst], sem))
        for c in copies:
            c.wait()

        iota = lax.iota(jnp.int32, _LANES)

        def body(i, carry):
            rows = i * _LANES + iota
            acc = jnp.zeros((_LANES,), jnp.float32)
            for d in range(D):
                dcol = jnp.full((_LANES,), d, jnp.int32)
                u = plsc.load_gather(urows, [rows, dcol])
                v = plsc.load_gather(irows, [rows, dcol])
                acc = acc + u * v
            plsc.store_scatter(outv, [rows], acc)
            return carry

        lax.fori_loop(0, bpw // _LANES, body, 0)
        pltpu.sync_copy(outv, out_hbm.at[pl.ds(base, bpw)])

    return kern


@jax.jit
def kernel(behavior, user_table, item_table):
    B = behavior.shape[0]
    uidx = behavior[:, 0].astype(jnp.int32).reshape(B // _CHUNK, _CHUNK)
    iidx = behavior[:, 1].astype(jnp.int32).reshape(B // _CHUNK, _CHUNK)
    return _make_kernel(B, item_table.shape[1])(uidx, iidx, user_table, item_table)


# speedup vs baseline: 1.4855x; 1.4855x over previous
import functools

import jax
import jax.numpy as jnp
from jax import lax
from jax.experimental import pallas as pl
from jax.experimental.pallas import tpu as pltpu
from jax.experimental.pallas import tpu_sc as plsc

_LANES = 16


@functools.lru_cache(maxsize=None)
def _make_kernel(B, D):
    info = plsc.get_sparse_core_info()
    NC, NS = info.num_cores, info.num_subcores
    NW = NC * NS
    bpw = B // NW
    nchunk = bpw // _LANES

    mesh = plsc.VectorSubcoreMesh(core_axis_name="c", subcore_axis_name="s")

    @functools.partial(
        pl.kernel,
        mesh=mesh,
        out_type=jax.ShapeDtypeStruct((B,), jnp.float32),
        compiler_params=pltpu.CompilerParams(needs_layout_passes=False),
        scratch_types=[
            pltpu.VMEM((bpw,), jnp.int32),
            pltpu.VMEM((bpw,), jnp.int32),
            pltpu.VMEM((4 * _LANES, D), jnp.float32),
            pltpu.VMEM((4 * _LANES, D), jnp.float32),
            pltpu.VMEM((bpw,), jnp.float32),
            pltpu.SemaphoreType.DMA,
        ],
    )
    def kern(uidx_hbm, iidx_hbm, ut_hbm, it_hbm, out_hbm,
             uix, iix, urows, irows, outv, sem):
        wid = lax.axis_index("s") * NC + lax.axis_index("c")
        base = wid * bpw
        pltpu.sync_copy(uidx_hbm.at[pl.ds(base, bpw)], uix)
        pltpu.sync_copy(iidx_hbm.at[pl.ds(base, bpw)], iix)

        iota = lax.iota(jnp.int32, _LANES)

        def fire(i):
            slot = lax.bitwise_and(i, 3)
            uvec = uix[pl.ds(i * _LANES, _LANES)]
            ivec = iix[pl.ds(i * _LANES, _LANES)]
            for j in range(_LANES):
                dst = pl.ds(slot * _LANES + j, 1)
                pltpu.async_copy(ut_hbm.at[pl.ds(uvec[j], 1)], urows.at[dst], sem)
                pltpu.async_copy(it_hbm.at[pl.ds(ivec[j], 1)], irows.at[dst], sem)

        def drain_compute(i):
            slot = lax.bitwise_and(i, 3)
            for j in range(_LANES):
                dst = pl.ds(slot * _LANES + j, 1)
                pltpu.make_async_copy(ut_hbm.at[pl.ds(0, 1)], urows.at[dst], sem).wait()
                pltpu.make_async_copy(ut_hbm.at[pl.ds(0, 1)], irows.at[dst], sem).wait()
            srows = slot * _LANES + iota
            acc = jnp.zeros((_LANES,), jnp.float32)
            for d in range(D):
                dcol = jnp.full((_LANES,), d, jnp.int32)
                u = plsc.load_gather(urows, [srows, dcol])
                v = plsc.load_gather(irows, [srows, dcol])
                acc = acc + u * v
            plsc.store_scatter(outv, [i * _LANES + iota], acc)

        def body(i, carry):
            fire(i)

            @pl.when(i >= 2)
            def _():
                drain_compute(i - 2)

            return carry

        lax.fori_loop(0, nchunk, body, 0)
        drain_compute(nchunk - 2)
        drain_compute(nchunk - 1)
        pltpu.sync_copy(outv, out_hbm.at[pl.ds(base, bpw)])

    return kern


@jax.jit
def kernel(behavior, user_table, item_table):
    uidx = behavior[:, 0].astype(jnp.int32)
    iidx = behavior[:, 1].astype(jnp.int32)
    return _make_kernel(behavior.shape[0], item_table.shape[1])(
        uidx, iidx, user_table, item_table
    )
